# speedup vs baseline: 104.5301x; 104.5301x over previous
import math

import jax
import jax.numpy as jnp
from jax.experimental import pallas as pl
from jax.experimental.pallas import tpu as pltpu

_NUM_BUCKETS = 32
_NUM_HEADS = 16
_Q = 2048
_K = 2048
_BQ = 128


def _bucket_of_n(n: int) -> int:
    if n < 16:
        return n
    return min(31, 16 + int(math.floor(16.0 * math.log2(n / 16.0))))


_BUCKETS = [_bucket_of_n(n) for n in range(32)]


def _bias_kernel(table_ref, out_ref, band_ref):
    h = pl.program_id(0)
    qi = pl.program_id(1)

    @pl.when(qi == 0)
    def _():
        dq = jax.lax.broadcasted_iota(jnp.int32, (_BQ, 2 * _BQ), 0)
        dk = jax.lax.broadcasted_iota(jnp.int32, (_BQ, 2 * _BQ), 1)
        n = jnp.clip(dq - dk + _BQ, 0, 31)
        acc = jnp.full((_BQ, 2 * _BQ), table_ref[0, 0, 0], jnp.float32)
        for b in range(1, 32):
            acc = jnp.where(n == b, table_ref[0, 0, _BUCKETS[b]], acc)
        band_ref[...] = acc

    c0 = table_ref[0, 0, 0]
    c31 = table_ref[0, 0, 31]

    col = jax.lax.broadcasted_iota(jnp.int32, (_BQ, _K), 1)
    out_ref[0, :, :] = jnp.where(col >= qi * _BQ, c0, c31)

    out_ref[0, :, pl.ds(qi * _BQ, _BQ)] = band_ref[:, _BQ:]

    @pl.when(qi > 0)
    def _():
        out_ref[0, :, pl.ds((qi - 1) * _BQ, _BQ)] = band_ref[:, :_BQ]


def kernel(query_len, key_len, relative_attention_bias):
    del query_len, key_len
    table = jnp.transpose(relative_attention_bias).reshape(_NUM_HEADS, 1, _NUM_BUCKETS)
    grid = (_NUM_HEADS, _Q // _BQ)
    return pl.pallas_call(
        _bias_kernel,
        grid=grid,
        in_specs=[
            pl.BlockSpec((1, 1, _NUM_BUCKETS), lambda h, qi: (h, 0, 0)),
        ],
        out_specs=pl.BlockSpec((1, _BQ, _K), lambda h, qi: (h, qi, 0)),
        out_shape=jax.ShapeDtypeStruct((_NUM_HEADS, _Q, _K), jnp.float32),
        scratch_shapes=[pltpu.VMEM((_BQ, 2 * _BQ), jnp.float32)],
    )(table)


# speedup vs baseline: 184.2840x; 1.7630x over previous
import math

import jax
import jax.numpy as jnp
from jax.experimental import pallas as pl
from jax.experimental.pallas import tpu as pltpu

_NUM_BUCKETS = 32
_NUM_HEADS = 16
_Q = 2048
_K = 2048
_BQ = 512
_SB = 128


def _bucket_of_n(n: int) -> int:
    if n < 16:
        return n
    return min(31, 16 + int(math.floor(16.0 * math.log2(n / 16.0))))


_BUCKETS = [_bucket_of_n(n) for n in range(32)]


def _bias_kernel(table_ref, out_ref, band_ref):
    h = pl.program_id(0)
    qi = pl.program_id(1)

    @pl.when(qi == 0)
    def _():
        dq = jax.lax.broadcasted_iota(jnp.int32, (_SB, 2 * _SB), 0)
        dk = jax.lax.broadcasted_iota(jnp.int32, (_SB, 2 * _SB), 1)
        n = jnp.clip(dq - dk + _SB, 0, 31)
        acc = jnp.full((_SB, 2 * _SB), table_ref[0, 0, 0], jnp.float32)
        for b in range(1, 32):
            acc = jnp.where(n == b, table_ref[0, 0, _BUCKETS[b]], acc)
        band_ref[...] = acc

    c0 = table_ref[0, 0, 0]
    c31 = table_ref[0, 0, 31]

    col = jax.lax.broadcasted_iota(jnp.int32, (_SB, _K), 1)
    n_sub = _BQ // _SB
    for r in range(n_sub):
        qb = qi * n_sub + r
        rows = slice(r * _SB, (r + 1) * _SB)
        out_ref[0, rows, :] = jnp.where(col >= qb * _SB, c0, c31)
        out_ref[0, rows, pl.ds(qb * _SB, _SB)] = band_ref[:, _SB:]
        if r > 0:
            out_ref[0, rows, pl.ds((qb - 1) * _SB, _SB)] = band_ref[:, :_SB]
        else:
            @pl.when(qi > 0)
            def _():
                out_ref[0, rows, pl.ds((qb - 1) * _SB, _SB)] = band_ref[:, :_SB]


def kernel(query_len, key_len, relative_attention_bias):
    del query_len, key_len
    table = jnp.transpose(relative_attention_bias).reshape(_NUM_HEADS, 1, _NUM_BUCKETS)
    grid = (_NUM_HEADS, _Q // _BQ)
    return pl.pallas_call(
        _bias_kernel,
        grid=grid,
        in_specs=[
            pl.BlockSpec((1, 1, _NUM_BUCKETS), lambda h, qi: (h, 0, 0)),
        ],
        out_specs=pl.BlockSpec((1, _BQ, _K), lambda h, qi: (h, qi, 0)),
        out_shape=jax.ShapeDtypeStruct((_NUM_HEADS, _Q, _K), jnp.float32),
        scratch_shapes=[pltpu.VMEM((_SB, 2 * _SB), jnp.float32)],
    )(table)


# speedup vs baseline: 195.5192x; 1.0610x over previous
import math

import jax
import jax.numpy as jnp
from jax.experimental import pallas as pl
from jax.experimental.pallas import tpu as pltpu

_NUM_BUCKETS = 32
_NUM_HEADS = 16
_Q = 2048
_K = 2048
_BQ = 1024
_SB = 128


def _bucket_of_n(n: int) -> int:
    if n < 16:
        return n
    return min(31, 16 + int(math.floor(16.0 * math.log2(n / 16.0))))


_BUCKETS = [_bucket_of_n(n) for n in range(32)]


def _bias_kernel(table_ref, out_ref, band_ref):
    h = pl.program_id(0)
    qi = pl.program_id(1)

    @pl.when(qi == 0)
    def _():
        dq = jax.lax.broadcasted_iota(jnp.int32, (_SB, 2 * _SB), 0)
        dk = jax.lax.broadcasted_iota(jnp.int32, (_SB, 2 * _SB), 1)
        n = jnp.clip(dq - dk + _SB, 0, 31)
        acc = jnp.full((_SB, 2 * _SB), table_ref[0, 0, 0], jnp.float32)
        for b in range(1, 32):
            acc = jnp.where(n == b, table_ref[0, 0, _BUCKETS[b]], acc)
        band_ref[...] = acc

    c0 = table_ref[0, 0, 0]
    c31 = table_ref[0, 0, 31]

    col = jax.lax.broadcasted_iota(jnp.int32, (_SB, _K), 1)
    n_sub = _BQ // _SB
    for r in range(n_sub):
        qb = qi * n_sub + r
        rows = slice(r * _SB, (r + 1) * _SB)
        out_ref[0, rows, :] = jnp.where(col >= qb * _SB, c0, c31)
        out_ref[0, rows, pl.ds(qb * _SB, _SB)] = band_ref[:, _SB:]
        if r > 0:
            out_ref[0, rows, pl.ds((qb - 1) * _SB, _SB)] = band_ref[:, :_SB]
        else:
            @pl.when(qi > 0)
            def _():
                out_ref[0, rows, pl.ds((qb - 1) * _SB, _SB)] = band_ref[:, :_SB]


def kernel(query_len, key_len, relative_attention_bias):
    del query_len, key_len
    table = jnp.transpose(relative_attention_bias).reshape(_NUM_HEADS, 1, _NUM_BUCKETS)
    grid = (_NUM_HEADS, _Q // _BQ)
    return pl.pallas_call(
        _bias_kernel,
        grid=grid,
        in_specs=[
            pl.BlockSpec((1, 1, _NUM_BUCKETS), lambda h, qi: (h, 0, 0)),
        ],
        out_specs=pl.BlockSpec((1, _BQ, _K), lambda h, qi: (h, qi, 0)),
        out_shape=jax.ShapeDtypeStruct((_NUM_HEADS, _Q, _K), jnp.float32),
        scratch_shapes=[pltpu.VMEM((_SB, 2 * _SB), jnp.float32)],
    )(table)
